# baseline (speedup 1.0000x reference)
import functools

import jax
import jax.numpy as jnp
from jax import lax
from jax.experimental import pallas as pl
from jax.experimental.pallas import tpu as pltpu
from jax.experimental.pallas import tpu_sc as plsc

_NC = 2
_NS = 16
_C = 128
_G = 8


def _sc_gather_scatter_add(x, src_p, dst_p, n_pad, ng1):
    ns, NGT, G, C = src_p.shape
    N, D = x.shape
    ng0 = NGT - ng1
    assert ng0 % 2 == 0 and ng1 % 2 == 0 and G >= 4
    rows_init = (N // (_NS * 8)) * 8
    init_tail = N - _NS * rows_init
    rows_out = n_pad // _NS
    mesh = plsc.VectorSubcoreMesh(core_axis_name="c", subcore_axis_name="s")

    @functools.partial(
        pl.kernel,
        mesh=mesh,
        out_type=jax.ShapeDtypeStruct((_NC, n_pad, D), jnp.float32),
        scratch_types=[
            pltpu.VMEM((G, C), jnp.int32),
            pltpu.VMEM((G, C), jnp.int32),
            pltpu.VMEM((G, C), jnp.int32),
            pltpu.VMEM((G, C), jnp.int32),
            pltpu.VMEM((C, D), jnp.float32),
            pltpu.VMEM((C, D), jnp.float32),
            pltpu.VMEM_SHARED((n_pad, D), jnp.float32),
            pltpu.SemaphoreType.DMA,
            pltpu.SemaphoreType.DMA,
            pltpu.SemaphoreType.DMA,
            pltpu.SemaphoreType.DMA,
            pltpu.SemaphoreType.DMA,
            pltpu.SemaphoreType.DMA,
        ],
    )
    def k(x_hbm, src_hbm, dst_hbm, out_hbm, sgA, sgB, dgA, dgB, rows0, rows1,
          acc, gsem0, gsem1, isemA, isemB, ssem0, ssem1):
        c = lax.axis_index("c")
        s = lax.axis_index("s")
        ng_c = jnp.where(c == 0, ng0, ng1)
        ngp = ng_c // 2
        gofs = jnp.where(c == 0, ng1, 0)
        pltpu.sync_copy(src_hbm.at[s, gofs], sgA)
        pltpu.sync_copy(dst_hbm.at[s, gofs], dgA)
        pltpu.async_copy(src_hbm.at[s, gofs + 1], sgB, isemB)
        pltpu.async_copy(dst_hbm.at[s, gofs + 1], dgB, isemB)
        pltpu.async_copy(x_hbm.at[sgA.at[0]], rows0, gsem0)
        pltpu.async_copy(x_hbm.at[sgA.at[1]], rows1, gsem1)
        pltpu.sync_copy(x_hbm.at[pl.ds(s * rows_init, rows_init)],
                        acc.at[pl.ds(s * rows_init, rows_init)])
        if init_tail:
            @pl.when(s == 0)
            def _():
                pltpu.sync_copy(
                    x_hbm.at[pl.ds(_NS * rows_init, init_tail)],
                    acc.at[pl.ds(_NS * rows_init, init_tail)])
        plsc.subcore_barrier()

        @pl.loop(0, ngp)
        def pair(p):
            gA = gofs + 2 * p
            gB = gA + 1
            more = 2 * p + 2 < ng_c

            @pl.loop(0, G - 2, step=2)
            def _(j):
                pltpu.make_async_copy(x_hbm.at[sgA.at[j]], rows0, gsem0).wait()
                pltpu.async_copy(rows0, acc.at[dgA.at[j]], ssem0, add=True)
                pltpu.make_async_copy(x_hbm.at[sgA.at[j + 1]], rows1,
                                      gsem1).wait()
                pltpu.async_copy(rows1, acc.at[dgA.at[j + 1]], ssem1, add=True)
                pltpu.make_async_copy(rows0, acc.at[dgA.at[j]], ssem0).wait()
                pltpu.async_copy(x_hbm.at[sgA.at[j + 2]], rows0, gsem0)
                pltpu.make_async_copy(rows1, acc.at[dgA.at[j + 1]],
                                      ssem1).wait()
                pltpu.async_copy(x_hbm.at[sgA.at[j + 3]], rows1, gsem1)

            pltpu.make_async_copy(src_hbm.at[s, gB], sgB, isemB).wait()
            pltpu.make_async_copy(dst_hbm.at[s, gB], dgB, isemB).wait()
            pltpu.make_async_copy(x_hbm.at[sgA.at[G - 2]], rows0, gsem0).wait()
            pltpu.async_copy(rows0, acc.at[dgA.at[G - 2]], ssem0, add=True)
            pltpu.make_async_copy(x_hbm.at[sgA.at[G - 1]], rows1, gsem1).wait()
            pltpu.async_copy(rows1, acc.at[dgA.at[G - 1]], ssem1, add=True)
            pltpu.make_async_copy(rows0, acc.at[dgA.at[G - 2]], ssem0).wait()
            pltpu.async_copy(x_hbm.at[sgB.at[0]], rows0, gsem0)
            pltpu.make_async_copy(rows1, acc.at[dgA.at[G - 1]], ssem1).wait()
            pltpu.async_copy(x_hbm.at[sgB.at[1]], rows1, gsem1)

            @pl.when(more)
            def _():
                pltpu.async_copy(src_hbm.at[s, gA + 2], sgA, isemA)
                pltpu.async_copy(dst_hbm.at[s, gA + 2], dgA, isemA)

            @pl.loop(0, G - 2, step=2)
            def _(j):
                pltpu.make_async_copy(x_hbm.at[sgB.at[j]], rows0, gsem0).wait()
                pltpu.async_copy(rows0, acc.at[dgB.at[j]], ssem0, add=True)
                pltpu.make_async_copy(x_hbm.at[sgB.at[j + 1]], rows1,
                                      gsem1).wait()
                pltpu.async_copy(rows1, acc.at[dgB.at[j + 1]], ssem1, add=True)
                pltpu.make_async_copy(rows0, acc.at[dgB.at[j]], ssem0).wait()
                pltpu.async_copy(x_hbm.at[sgB.at[j + 2]], rows0, gsem0)
                pltpu.make_async_copy(rows1, acc.at[dgB.at[j + 1]],
                                      ssem1).wait()
                pltpu.async_copy(x_hbm.at[sgB.at[j + 3]], rows1, gsem1)

            @pl.when(more)
            def _():
                pltpu.make_async_copy(src_hbm.at[s, gA + 2], sgA, isemA).wait()
                pltpu.make_async_copy(dst_hbm.at[s, gA + 2], dgA, isemA).wait()

            pltpu.make_async_copy(x_hbm.at[sgB.at[G - 2]], rows0, gsem0).wait()
            pltpu.async_copy(rows0, acc.at[dgB.at[G - 2]], ssem0, add=True)
            pltpu.make_async_copy(x_hbm.at[sgB.at[G - 1]], rows1, gsem1).wait()
            pltpu.async_copy(rows1, acc.at[dgB.at[G - 1]], ssem1, add=True)

            @pl.when(more)
            def _():
                pltpu.make_async_copy(rows0, acc.at[dgB.at[G - 2]],
                                      ssem0).wait()
                pltpu.async_copy(x_hbm.at[sgA.at[0]], rows0, gsem0)
                pltpu.make_async_copy(rows1, acc.at[dgB.at[G - 1]],
                                      ssem1).wait()
                pltpu.async_copy(x_hbm.at[sgA.at[1]], rows1, gsem1)

            @pl.when(2 * p + 3 < ng_c)
            def _():
                pltpu.async_copy(src_hbm.at[s, gB + 2], sgB, isemB)
                pltpu.async_copy(dst_hbm.at[s, gB + 2], dgB, isemB)

        pltpu.make_async_copy(rows0, acc.at[dgB.at[G - 2]], ssem0).wait()
        pltpu.make_async_copy(rows1, acc.at[dgB.at[G - 1]], ssem1).wait()
        plsc.subcore_barrier()
        pltpu.sync_copy(acc.at[pl.ds(s * rows_out, rows_out)],
                        out_hbm.at[c, pl.ds(s * rows_out, rows_out)])

    return k(x, src_p, dst_p)


def _mlp(agg, x, Wa, ba, Wb, bb, block_rows):
    N, D = x.shape
    grid = N // block_rows

    def body(agg_ref, x_ref, wa, ba_r, wb, bb_r, o_ref):
        g = agg_ref[0] + agg_ref[1] - x_ref[...]
        h1 = jnp.maximum(
            jnp.dot(g, wa[...], preferred_element_type=jnp.float32)
            + ba_r[...], 0.0)
        o_ref[...] = (jnp.dot(h1, wb[...], preferred_element_type=jnp.float32)
                      + bb_r[...])

    return pl.pallas_call(
        body,
        grid=(grid,),
        in_specs=[
            pl.BlockSpec((_NC, block_rows, D), lambda i: (0, i, 0)),
            pl.BlockSpec((block_rows, D), lambda i: (i, 0)),
            pl.BlockSpec((D, D), lambda i: (0, 0)),
            pl.BlockSpec((1, D), lambda i: (0, 0)),
            pl.BlockSpec((D, D), lambda i: (0, 0)),
            pl.BlockSpec((1, D), lambda i: (0, 0)),
        ],
        out_specs=pl.BlockSpec((block_rows, D), lambda i: (i, 0)),
        out_shape=jax.ShapeDtypeStruct((N, D), jnp.float32),
    )(agg, x, Wa, ba.reshape(1, D), Wb, bb.reshape(1, D))


def _mlp_pool_classify(agg, x, Wa, ba, Wb, bb, batch3, Wc1, bc1, Wc2, bc2,
                       num_graphs, block_rows):
    N, D = x.shape
    n_classes = Wc2.shape[1]
    grid = N // block_rows

    def body(agg_ref, x_ref, b_ref, wa, ba_r, wb, bb_r, wc1, bc1_r, wc2,
             bc2_r, o_ref, acc_ref):
        i = pl.program_id(0)

        @pl.when(i == 0)
        def _():
            acc_ref[...] = jnp.zeros_like(acc_ref)

        g = agg_ref[0] + agg_ref[1] - x_ref[...]
        h1 = jnp.maximum(
            jnp.dot(g, wa[...], preferred_element_type=jnp.float32)
            + ba_r[...], 0.0)
        h3 = (jnp.dot(h1, wb[...], preferred_element_type=jnp.float32)
              + bb_r[...])
        b = b_ref[0, 0, :]
        onehot = (b[:, None] == lax.broadcasted_iota(
            jnp.int32, (block_rows, num_graphs), 1)).astype(jnp.float32)
        acc_ref[...] += lax.dot_general(
            onehot, h3, (((0,), (0,)), ((), ())),
            preferred_element_type=jnp.float32)

        @pl.when(i == grid - 1)
        def _():
            t = jnp.maximum(
                jnp.dot(acc_ref[...], wc1[...],
                        preferred_element_type=jnp.float32) + bc1_r[...], 0.0)
            o_ref[...] = (jnp.dot(t, wc2[...],
                                  preferred_element_type=jnp.float32)
                          + bc2_r[...])

    return pl.pallas_call(
        body,
        grid=(grid,),
        in_specs=[
            pl.BlockSpec((_NC, block_rows, D), lambda i: (0, i, 0)),
            pl.BlockSpec((block_rows, D), lambda i: (i, 0)),
            pl.BlockSpec((1, 1, block_rows), lambda i: (i, 0, 0)),
            pl.BlockSpec((D, D), lambda i: (0, 0)),
            pl.BlockSpec((1, D), lambda i: (0, 0)),
            pl.BlockSpec((D, D), lambda i: (0, 0)),
            pl.BlockSpec((1, D), lambda i: (0, 0)),
            pl.BlockSpec((D, D), lambda i: (0, 0)),
            pl.BlockSpec((1, D), lambda i: (0, 0)),
            pl.BlockSpec((D, n_classes), lambda i: (0, 0)),
            pl.BlockSpec((1, n_classes), lambda i: (0, 0)),
        ],
        out_specs=pl.BlockSpec((num_graphs, n_classes), lambda i: (0, 0)),
        out_shape=jax.ShapeDtypeStruct((num_graphs, n_classes), jnp.float32),
        scratch_shapes=[pltpu.VMEM((num_graphs, D), jnp.float32)],
    )(agg, x, batch3, Wa, ba.reshape(1, D), Wb, bb.reshape(1, D),
      Wc1, bc1.reshape(1, D), Wc2, bc2.reshape(1, n_classes))


def kernel(x, edge_index, batch, W1a, b1a, W1b, b1b, W2a, b2a, W2b, b2b,
           W3a, b3a, W3b, b3b, Wc1, bc1, Wc2, bc2):
    N, D = x.shape
    E = edge_index.shape[1]
    num_graphs = 64
    per_slab = -(-E // _NS)
    chunks_per_slab = -(-per_slab // _C)
    NGT = -(-chunks_per_slab // _G)
    ng1 = NGT // 2
    ng1 = ng1 - (ng1 % 2)
    slab_cap = NGT * _G * _C
    n_pad = -(-(N + 1) // (_NS * 8)) * _NS * 8

    src = edge_index[0]
    dst = edge_index[1]
    if E % _NS == 0:
        pad = slab_cap - E // _NS
        fill = (jnp.arange(_NS * pad, dtype=jnp.int32) % N).reshape(_NS, pad)
        src_p = jnp.concatenate([src.reshape(_NS, E // _NS), fill], axis=1)
        dst_p = jnp.pad(dst.reshape(_NS, E // _NS), ((0, 0), (0, pad)),
                        constant_values=N)
    else:
        pad = _NS * slab_cap - E
        fill = jnp.arange(pad, dtype=jnp.int32) % N
        src_p = jnp.concatenate([src, fill])
        dst_p = jnp.concatenate([dst, jnp.full((pad,), N, jnp.int32)])
    src_p = src_p.reshape(_NS, NGT, _G, _C)
    dst_p = dst_p.reshape(_NS, NGT, _G, _C)

    block_rows = 1000
    batch3 = batch.reshape(N // block_rows, 1, block_rows)

    h = x
    for (Wa, ba, Wb, bb) in ((W1a, b1a, W1b, b1b), (W2a, b2a, W2b, b2b)):
        agg = _sc_gather_scatter_add(h, src_p, dst_p, n_pad, ng1)
        h = _mlp(agg, h, Wa, ba, Wb, bb, block_rows)

    agg = _sc_gather_scatter_add(h, src_p, dst_p, n_pad, ng1)
    return _mlp_pool_classify(agg, h, W3a, b3a, W3b, b3b, batch3,
                              Wc1, bc1, Wc2, bc2, num_graphs, block_rows)


# speedup vs baseline: 1.2726x; 1.2726x over previous
import functools

import jax
import jax.numpy as jnp
from jax import lax
from jax.experimental import pallas as pl
from jax.experimental.pallas import tpu as pltpu
from jax.experimental.pallas import tpu_sc as plsc

_NC = 2
_NS = 16
_C = 128
_G = 8


def _sc_gather_scatter_add(x, src_p, dst_p, n_pad, ng1):
    ns, NGT, G, C = src_p.shape
    N, D = x.shape
    ng0 = NGT - ng1
    assert ng0 % 2 == 0 and ng1 % 2 == 0 and G >= 4
    rows_init = (N // (_NS * 8)) * 8
    init_tail = N - _NS * rows_init
    rows_out = n_pad // _NS
    mesh = plsc.VectorSubcoreMesh(core_axis_name="c", subcore_axis_name="s")

    @functools.partial(
        pl.kernel,
        mesh=mesh,
        out_type=jax.ShapeDtypeStruct((_NC, n_pad, D), jnp.float32),
        scratch_types=[
            pltpu.VMEM((G, C), jnp.int32),
            pltpu.VMEM((G, C), jnp.int32),
            pltpu.VMEM((G, C), jnp.int32),
            pltpu.VMEM((G, C), jnp.int32),
            pltpu.VMEM((C, D), jnp.float32),
            pltpu.VMEM((C, D), jnp.float32),
            pltpu.VMEM_SHARED((n_pad, D), jnp.float32),
            pltpu.SemaphoreType.DMA,
            pltpu.SemaphoreType.DMA,
            pltpu.SemaphoreType.DMA,
            pltpu.SemaphoreType.DMA,
        ],
    )
    def k(x_hbm, src_hbm, dst_hbm, out_hbm, sgA, sgB, dgA, dgB, rows0, rows1,
          acc, gsem0, gsem1, isemA, isemB):
        c = lax.axis_index("c")
        s = lax.axis_index("s")
        ng_c = jnp.where(c == 0, ng0, ng1)
        ngp = ng_c // 2
        gofs = jnp.where(c == 0, ng1, 0)
        pltpu.sync_copy(src_hbm.at[s, gofs], sgA)
        pltpu.sync_copy(dst_hbm.at[s, gofs], dgA)
        pltpu.async_copy(src_hbm.at[s, gofs + 1], sgB, isemB)
        pltpu.async_copy(dst_hbm.at[s, gofs + 1], dgB, isemB)
        pltpu.async_copy(x_hbm.at[sgA.at[0]], rows0, gsem0)
        pltpu.async_copy(x_hbm.at[sgA.at[1]], rows1, gsem1)
        pltpu.sync_copy(x_hbm.at[pl.ds(s * rows_init, rows_init)],
                        acc.at[pl.ds(s * rows_init, rows_init)])
        if init_tail:
            @pl.when(s == 0)
            def _():
                pltpu.sync_copy(
                    x_hbm.at[pl.ds(_NS * rows_init, init_tail)],
                    acc.at[pl.ds(_NS * rows_init, init_tail)])
        plsc.subcore_barrier()

        @pl.loop(0, ngp)
        def pair(p):
            gA = gofs + 2 * p
            gB = gA + 1
            more = 2 * p + 2 < ng_c

            @pl.loop(0, G - 2, step=2)
            def _(j):
                pltpu.make_async_copy(x_hbm.at[sgA.at[j]], rows0, gsem0).wait()
                pltpu.sync_copy(rows0, acc.at[dgA.at[j]], add=True)
                pltpu.async_copy(x_hbm.at[sgA.at[j + 2]], rows0, gsem0)
                pltpu.make_async_copy(x_hbm.at[sgA.at[j + 1]], rows1,
                                      gsem1).wait()
                pltpu.sync_copy(rows1, acc.at[dgA.at[j + 1]], add=True)
                pltpu.async_copy(x_hbm.at[sgA.at[j + 3]], rows1, gsem1)

            pltpu.make_async_copy(src_hbm.at[s, gB], sgB, isemB).wait()
            pltpu.make_async_copy(dst_hbm.at[s, gB], dgB, isemB).wait()
            pltpu.make_async_copy(x_hbm.at[sgA.at[G - 2]], rows0, gsem0).wait()
            pltpu.sync_copy(rows0, acc.at[dgA.at[G - 2]], add=True)
            pltpu.async_copy(x_hbm.at[sgB.at[0]], rows0, gsem0)
            pltpu.make_async_copy(x_hbm.at[sgA.at[G - 1]], rows1, gsem1).wait()
            pltpu.sync_copy(rows1, acc.at[dgA.at[G - 1]], add=True)
            pltpu.async_copy(x_hbm.at[sgB.at[1]], rows1, gsem1)

            @pl.when(more)
            def _():
                pltpu.async_copy(src_hbm.at[s, gA + 2], sgA, isemA)
                pltpu.async_copy(dst_hbm.at[s, gA + 2], dgA, isemA)

            @pl.loop(0, G - 2, step=2)
            def _(j):
                pltpu.make_async_copy(x_hbm.at[sgB.at[j]], rows0, gsem0).wait()
                pltpu.sync_copy(rows0, acc.at[dgB.at[j]], add=True)
                pltpu.async_copy(x_hbm.at[sgB.at[j + 2]], rows0, gsem0)
                pltpu.make_async_copy(x_hbm.at[sgB.at[j + 1]], rows1,
                                      gsem1).wait()
                pltpu.sync_copy(rows1, acc.at[dgB.at[j + 1]], add=True)
                pltpu.async_copy(x_hbm.at[sgB.at[j + 3]], rows1, gsem1)

            @pl.when(more)
            def _():
                pltpu.make_async_copy(src_hbm.at[s, gA + 2], sgA, isemA).wait()
                pltpu.make_async_copy(dst_hbm.at[s, gA + 2], dgA, isemA).wait()

            pltpu.make_async_copy(x_hbm.at[sgB.at[G - 2]], rows0, gsem0).wait()
            pltpu.sync_copy(rows0, acc.at[dgB.at[G - 2]], add=True)

            @pl.when(more)
            def _():
                pltpu.async_copy(x_hbm.at[sgA.at[0]], rows0, gsem0)

            pltpu.make_async_copy(x_hbm.at[sgB.at[G - 1]], rows1, gsem1).wait()
            pltpu.sync_copy(rows1, acc.at[dgB.at[G - 1]], add=True)

            @pl.when(more)
            def _():
                pltpu.async_copy(x_hbm.at[sgA.at[1]], rows1, gsem1)

            @pl.when(2 * p + 3 < ng_c)
            def _():
                pltpu.async_copy(src_hbm.at[s, gB + 2], sgB, isemB)
                pltpu.async_copy(dst_hbm.at[s, gB + 2], dgB, isemB)

        plsc.subcore_barrier()
        pltpu.sync_copy(acc.at[pl.ds(s * rows_out, rows_out)],
                        out_hbm.at[c, pl.ds(s * rows_out, rows_out)])

    return k(x, src_p, dst_p)


def _mlp(agg, x, Wa, ba, Wb, bb, block_rows):
    N, D = x.shape
    grid = N // block_rows

    def body(agg_ref, x_ref, wa, ba_r, wb, bb_r, o_ref):
        g = agg_ref[0] + agg_ref[1] - x_ref[...]
        h1 = jnp.maximum(
            jnp.dot(g, wa[...], preferred_element_type=jnp.float32)
            + ba_r[...], 0.0)
        o_ref[...] = (jnp.dot(h1, wb[...], preferred_element_type=jnp.float32)
                      + bb_r[...])

    return pl.pallas_call(
        body,
        grid=(grid,),
        in_specs=[
            pl.BlockSpec((_NC, block_rows, D), lambda i: (0, i, 0)),
            pl.BlockSpec((block_rows, D), lambda i: (i, 0)),
            pl.BlockSpec((D, D), lambda i: (0, 0)),
            pl.BlockSpec((1, D), lambda i: (0, 0)),
            pl.BlockSpec((D, D), lambda i: (0, 0)),
            pl.BlockSpec((1, D), lambda i: (0, 0)),
        ],
        out_specs=pl.BlockSpec((block_rows, D), lambda i: (i, 0)),
        out_shape=jax.ShapeDtypeStruct((N, D), jnp.float32),
    )(agg, x, Wa, ba.reshape(1, D), Wb, bb.reshape(1, D))


def _mlp_pool_classify(agg, x, Wa, ba, Wb, bb, batch3, Wc1, bc1, Wc2, bc2,
                       num_graphs, block_rows):
    N, D = x.shape
    n_classes = Wc2.shape[1]
    grid = N // block_rows

    def body(agg_ref, x_ref, b_ref, wa, ba_r, wb, bb_r, wc1, bc1_r, wc2,
             bc2_r, o_ref, acc_ref):
        i = pl.program_id(0)

        @pl.when(i == 0)
        def _():
            acc_ref[...] = jnp.zeros_like(acc_ref)

        g = agg_ref[0] + agg_ref[1] - x_ref[...]
        h1 = jnp.maximum(
            jnp.dot(g, wa[...], preferred_element_type=jnp.float32)
            + ba_r[...], 0.0)
        h3 = (jnp.dot(h1, wb[...], preferred_element_type=jnp.float32)
              + bb_r[...])
        b = b_ref[0, 0, :]
        onehot = (b[:, None] == lax.broadcasted_iota(
            jnp.int32, (block_rows, num_graphs), 1)).astype(jnp.float32)
        acc_ref[...] += lax.dot_general(
            onehot, h3, (((0,), (0,)), ((), ())),
            preferred_element_type=jnp.float32)

        @pl.when(i == grid - 1)
        def _():
            t = jnp.maximum(
                jnp.dot(acc_ref[...], wc1[...],
                        preferred_element_type=jnp.float32) + bc1_r[...], 0.0)
            o_ref[...] = (jnp.dot(t, wc2[...],
                                  preferred_element_type=jnp.float32)
                          + bc2_r[...])

    return pl.pallas_call(
        body,
        grid=(grid,),
        in_specs=[
            pl.BlockSpec((_NC, block_rows, D), lambda i: (0, i, 0)),
            pl.BlockSpec((block_rows, D), lambda i: (i, 0)),
            pl.BlockSpec((1, 1, block_rows), lambda i: (i, 0, 0)),
            pl.BlockSpec((D, D), lambda i: (0, 0)),
            pl.BlockSpec((1, D), lambda i: (0, 0)),
            pl.BlockSpec((D, D), lambda i: (0, 0)),
            pl.BlockSpec((1, D), lambda i: (0, 0)),
            pl.BlockSpec((D, D), lambda i: (0, 0)),
            pl.BlockSpec((1, D), lambda i: (0, 0)),
            pl.BlockSpec((D, n_classes), lambda i: (0, 0)),
            pl.BlockSpec((1, n_classes), lambda i: (0, 0)),
        ],
        out_specs=pl.BlockSpec((num_graphs, n_classes), lambda i: (0, 0)),
        out_shape=jax.ShapeDtypeStruct((num_graphs, n_classes), jnp.float32),
        scratch_shapes=[pltpu.VMEM((num_graphs, D), jnp.float32)],
    )(agg, x, batch3, Wa, ba.reshape(1, D), Wb, bb.reshape(1, D),
      Wc1, bc1.reshape(1, D), Wc2, bc2.reshape(1, n_classes))


def kernel(x, edge_index, batch, W1a, b1a, W1b, b1b, W2a, b2a, W2b, b2b,
           W3a, b3a, W3b, b3b, Wc1, bc1, Wc2, bc2):
    N, D = x.shape
    E = edge_index.shape[1]
    num_graphs = 64
    per_slab = -(-E // _NS)
    chunks_per_slab = -(-per_slab // _C)
    NGT = -(-chunks_per_slab // _G)
    ng1 = NGT // 2
    ng1 = ng1 - (ng1 % 2)
    slab_cap = NGT * _G * _C
    n_pad = -(-(N + 1) // (_NS * 8)) * _NS * 8

    src = edge_index[0]
    dst = edge_index[1]
    if E % _NS == 0:
        pad = slab_cap - E // _NS
        fill = (jnp.arange(_NS * pad, dtype=jnp.int32) % N).reshape(_NS, pad)
        src_p = jnp.concatenate([src.reshape(_NS, E // _NS), fill], axis=1)
        dst_p = jnp.pad(dst.reshape(_NS, E // _NS), ((0, 0), (0, pad)),
                        constant_values=N)
    else:
        pad = _NS * slab_cap - E
        fill = jnp.arange(pad, dtype=jnp.int32) % N
        src_p = jnp.concatenate([src, fill])
        dst_p = jnp.concatenate([dst, jnp.full((pad,), N, jnp.int32)])
    src_p = src_p.reshape(_NS, NGT, _G, _C)
    dst_p = dst_p.reshape(_NS, NGT, _G, _C)

    block_rows = 1000
    batch3 = batch.reshape(N // block_rows, 1, block_rows)

    h = x
    for (Wa, ba, Wb, bb) in ((W1a, b1a, W1b, b1b), (W2a, b2a, W2b, b2b)):
        agg = _sc_gather_scatter_add(h, src_p, dst_p, n_pad, ng1)
        h = _mlp(agg, h, Wa, ba, Wb, bb, block_rows)

    agg = _sc_gather_scatter_add(h, src_p, dst_p, n_pad, ng1)
    return _mlp_pool_classify(agg, h, W3a, b3a, W3b, b3b, batch3,
                              Wc1, bc1, Wc2, bc2, num_graphs, block_rows)


# speedup vs baseline: 1.3031x; 1.0239x over previous
import functools

import jax
import jax.numpy as jnp
from jax import lax
from jax.experimental import pallas as pl
from jax.experimental.pallas import tpu as pltpu
from jax.experimental.pallas import tpu_sc as plsc

_NC = 2
_NS = 16
_C = 128
_G = 8


def _sc_gather_scatter_add(x, src_p, dst_p, n_pad, ng1):
    ns, NGT, G, C = src_p.shape
    N, D = x.shape
    ng0 = NGT - ng1
    assert ng0 % 2 == 0 and ng1 % 2 == 0 and G >= 4
    rows_init = (N // (_NS * 8)) * 8
    init_tail = N - _NS * rows_init
    rows_out = n_pad // _NS
    mesh = plsc.VectorSubcoreMesh(core_axis_name="c", subcore_axis_name="s")

    @functools.partial(
        pl.kernel,
        mesh=mesh,
        out_type=jax.ShapeDtypeStruct((_NC, n_pad, D), jnp.float32),
        scratch_types=[
            pltpu.VMEM((G, C), jnp.int32),
            pltpu.VMEM((G, C), jnp.int32),
            pltpu.VMEM((G, C), jnp.int32),
            pltpu.VMEM((G, C), jnp.int32),
            pltpu.VMEM((C, D), jnp.float32),
            pltpu.VMEM((C, D), jnp.float32),
            pltpu.VMEM_SHARED((n_pad, D), jnp.float32),
            pltpu.SemaphoreType.DMA,
            pltpu.SemaphoreType.DMA,
            pltpu.SemaphoreType.DMA,
            pltpu.SemaphoreType.DMA,
        ],
    )
    def k(x_hbm, src_hbm, dst_hbm, out_hbm, sgA, sgB, dgA, dgB, rows0, rows1,
          acc, gsem0, gsem1, isemA, isemB):
        c = lax.axis_index("c")
        s = lax.axis_index("s")
        ng_c = jnp.where(c == 0, ng0, ng1)
        ngp = ng_c // 2
        gofs = jnp.where(c == 0, ng1, 0)
        pltpu.sync_copy(src_hbm.at[s, gofs], sgA)
        pltpu.sync_copy(dst_hbm.at[s, gofs], dgA)
        pltpu.async_copy(src_hbm.at[s, gofs + 1], sgB, isemB)
        pltpu.async_copy(dst_hbm.at[s, gofs + 1], dgB, isemB)
        pltpu.async_copy(x_hbm.at[sgA.at[0]], rows0, gsem0)
        pltpu.async_copy(x_hbm.at[sgA.at[1]], rows1, gsem1)
        pltpu.sync_copy(x_hbm.at[pl.ds(s * rows_init, rows_init)],
                        acc.at[pl.ds(s * rows_init, rows_init)])
        if init_tail:
            @pl.when(s == 0)
            def _():
                pltpu.sync_copy(
                    x_hbm.at[pl.ds(_NS * rows_init, init_tail)],
                    acc.at[pl.ds(_NS * rows_init, init_tail)])
        plsc.subcore_barrier()

        @pl.loop(0, ngp)
        def pair(p):
            gA = gofs + 2 * p
            gB = gA + 1
            more = 2 * p + 2 < ng_c

            @pl.loop(0, G - 2, step=2)
            def _(j):
                pltpu.make_async_copy(x_hbm.at[sgA.at[j]], rows0, gsem0).wait()
                pltpu.sync_copy(rows0, acc.at[dgA.at[j]], add=True)
                pltpu.async_copy(x_hbm.at[sgA.at[j + 2]], rows0, gsem0)
                pltpu.make_async_copy(x_hbm.at[sgA.at[j + 1]], rows1,
                                      gsem1).wait()
                pltpu.sync_copy(rows1, acc.at[dgA.at[j + 1]], add=True)
                pltpu.async_copy(x_hbm.at[sgA.at[j + 3]], rows1, gsem1)

            pltpu.make_async_copy(src_hbm.at[s, gB], sgB, isemB).wait()
            pltpu.make_async_copy(dst_hbm.at[s, gB], dgB, isemB).wait()
            pltpu.make_async_copy(x_hbm.at[sgA.at[G - 2]], rows0, gsem0).wait()
            pltpu.sync_copy(rows0, acc.at[dgA.at[G - 2]], add=True)
            pltpu.async_copy(x_hbm.at[sgB.at[0]], rows0, gsem0)
            pltpu.make_async_copy(x_hbm.at[sgA.at[G - 1]], rows1, gsem1).wait()
            pltpu.sync_copy(rows1, acc.at[dgA.at[G - 1]], add=True)
            pltpu.async_copy(x_hbm.at[sgB.at[1]], rows1, gsem1)

            @pl.when(more)
            def _():
                pltpu.async_copy(src_hbm.at[s, gA + 2], sgA, isemA)
                pltpu.async_copy(dst_hbm.at[s, gA + 2], dgA, isemA)

            @pl.loop(0, G - 2, step=2)
            def _(j):
                pltpu.make_async_copy(x_hbm.at[sgB.at[j]], rows0, gsem0).wait()
                pltpu.sync_copy(rows0, acc.at[dgB.at[j]], add=True)
                pltpu.async_copy(x_hbm.at[sgB.at[j + 2]], rows0, gsem0)
                pltpu.make_async_copy(x_hbm.at[sgB.at[j + 1]], rows1,
                                      gsem1).wait()
                pltpu.sync_copy(rows1, acc.at[dgB.at[j + 1]], add=True)
                pltpu.async_copy(x_hbm.at[sgB.at[j + 3]], rows1, gsem1)

            @pl.when(more)
            def _():
                pltpu.make_async_copy(src_hbm.at[s, gA + 2], sgA, isemA).wait()
                pltpu.make_async_copy(dst_hbm.at[s, gA + 2], dgA, isemA).wait()

            pltpu.make_async_copy(x_hbm.at[sgB.at[G - 2]], rows0, gsem0).wait()
            pltpu.sync_copy(rows0, acc.at[dgB.at[G - 2]], add=True)

            @pl.when(more)
            def _():
                pltpu.async_copy(x_hbm.at[sgA.at[0]], rows0, gsem0)

            pltpu.make_async_copy(x_hbm.at[sgB.at[G - 1]], rows1, gsem1).wait()
            pltpu.sync_copy(rows1, acc.at[dgB.at[G - 1]], add=True)

            @pl.when(more)
            def _():
                pltpu.async_copy(x_hbm.at[sgA.at[1]], rows1, gsem1)

            @pl.when(2 * p + 3 < ng_c)
            def _():
                pltpu.async_copy(src_hbm.at[s, gB + 2], sgB, isemB)
                pltpu.async_copy(dst_hbm.at[s, gB + 2], dgB, isemB)

        plsc.subcore_barrier()
        pltpu.sync_copy(acc.at[pl.ds(s * rows_out, rows_out)],
                        out_hbm.at[c, pl.ds(s * rows_out, rows_out)])

    return k(x, src_p, dst_p)


def _mlp(agg, x, Wa, ba, Wb, bb, block_rows):
    N, D = x.shape
    grid = N // block_rows

    def body(agg_ref, x_ref, wa, ba_r, wb, bb_r, o_ref):
        g = agg_ref[0] + agg_ref[1] - x_ref[...]
        h1 = jnp.maximum(
            jnp.dot(g, wa[...], preferred_element_type=jnp.float32)
            + ba_r[...], 0.0)
        o_ref[...] = (jnp.dot(h1, wb[...], preferred_element_type=jnp.float32)
                      + bb_r[...])

    return pl.pallas_call(
        body,
        grid=(grid,),
        in_specs=[
            pl.BlockSpec((_NC, block_rows, D), lambda i: (0, i, 0)),
            pl.BlockSpec((block_rows, D), lambda i: (i, 0)),
            pl.BlockSpec((D, D), lambda i: (0, 0)),
            pl.BlockSpec((1, D), lambda i: (0, 0)),
            pl.BlockSpec((D, D), lambda i: (0, 0)),
            pl.BlockSpec((1, D), lambda i: (0, 0)),
        ],
        out_specs=pl.BlockSpec((block_rows, D), lambda i: (i, 0)),
        out_shape=jax.ShapeDtypeStruct((N, D), jnp.float32),
    )(agg, x, Wa, ba.reshape(1, D), Wb, bb.reshape(1, D))


def _mlp_pool_classify(agg, x, Wa, ba, Wb, bb, batch3, Wc1, bc1, Wc2, bc2,
                       num_graphs, block_rows):
    N, D = x.shape
    n_classes = Wc2.shape[1]
    grid = N // block_rows

    def body(agg_ref, x_ref, b_ref, wa, ba_r, wb, bb_r, wc1, bc1_r, wc2,
             bc2_r, o_ref, acc_ref):
        i = pl.program_id(0)

        @pl.when(i == 0)
        def _():
            acc_ref[...] = jnp.zeros_like(acc_ref)

        g = agg_ref[0] + agg_ref[1] - x_ref[...]
        h1 = jnp.maximum(
            jnp.dot(g, wa[...], preferred_element_type=jnp.float32)
            + ba_r[...], 0.0)
        h3 = (jnp.dot(h1, wb[...], preferred_element_type=jnp.float32)
              + bb_r[...])
        b = b_ref[0, 0, :]
        onehot = (b[:, None] == lax.broadcasted_iota(
            jnp.int32, (block_rows, num_graphs), 1)).astype(jnp.float32)
        acc_ref[...] += lax.dot_general(
            onehot, h3, (((0,), (0,)), ((), ())),
            preferred_element_type=jnp.float32)

        @pl.when(i == grid - 1)
        def _():
            t = jnp.maximum(
                jnp.dot(acc_ref[...], wc1[...],
                        preferred_element_type=jnp.float32) + bc1_r[...], 0.0)
            o_ref[...] = (jnp.dot(t, wc2[...],
                                  preferred_element_type=jnp.float32)
                          + bc2_r[...])

    return pl.pallas_call(
        body,
        grid=(grid,),
        in_specs=[
            pl.BlockSpec((_NC, block_rows, D), lambda i: (0, i, 0)),
            pl.BlockSpec((block_rows, D), lambda i: (i, 0)),
            pl.BlockSpec((1, 1, block_rows), lambda i: (i, 0, 0)),
            pl.BlockSpec((D, D), lambda i: (0, 0)),
            pl.BlockSpec((1, D), lambda i: (0, 0)),
            pl.BlockSpec((D, D), lambda i: (0, 0)),
            pl.BlockSpec((1, D), lambda i: (0, 0)),
            pl.BlockSpec((D, D), lambda i: (0, 0)),
            pl.BlockSpec((1, D), lambda i: (0, 0)),
            pl.BlockSpec((D, n_classes), lambda i: (0, 0)),
            pl.BlockSpec((1, n_classes), lambda i: (0, 0)),
        ],
        out_specs=pl.BlockSpec((num_graphs, n_classes), lambda i: (0, 0)),
        out_shape=jax.ShapeDtypeStruct((num_graphs, n_classes), jnp.float32),
        scratch_shapes=[pltpu.VMEM((num_graphs, D), jnp.float32)],
    )(agg, x, batch3, Wa, ba.reshape(1, D), Wb, bb.reshape(1, D),
      Wc1, bc1.reshape(1, D), Wc2, bc2.reshape(1, n_classes))


def kernel(x, edge_index, batch, W1a, b1a, W1b, b1b, W2a, b2a, W2b, b2b,
           W3a, b3a, W3b, b3b, Wc1, bc1, Wc2, bc2):
    N, D = x.shape
    E = edge_index.shape[1]
    num_graphs = 64
    per_slab = -(-E // _NS)
    chunks_per_slab = -(-per_slab // _C)
    NGT = -(-chunks_per_slab // _G)
    ng1 = NGT // 2
    ng1 = ng1 - (ng1 % 2)
    slab_cap = NGT * _G * _C
    n_pad = -(-(N + 1) // (_NS * 8)) * _NS * 8

    src = edge_index[0]
    dst = edge_index[1]
    if E % _NS == 0:
        pad = slab_cap - E // _NS
        fill = (jnp.arange(_NS * pad, dtype=jnp.int32) % N).reshape(_NS, pad)
        src_p = jnp.concatenate([src.reshape(_NS, E // _NS), fill], axis=1)
        dst_p = jnp.pad(dst.reshape(_NS, E // _NS), ((0, 0), (0, pad)),
                        constant_values=N)
    else:
        pad = _NS * slab_cap - E
        fill = jnp.arange(pad, dtype=jnp.int32) % N
        src_p = jnp.concatenate([src, fill])
        dst_p = jnp.concatenate([dst, jnp.full((pad,), N, jnp.int32)])
    src_p = src_p.reshape(_NS, NGT, _G, _C)
    dst_p = dst_p.reshape(_NS, NGT, _G, _C)

    block_rows = 2000
    batch3 = batch.reshape(N // block_rows, 1, block_rows)

    h = x
    for (Wa, ba, Wb, bb) in ((W1a, b1a, W1b, b1b), (W2a, b2a, W2b, b2b)):
        agg = _sc_gather_scatter_add(h, src_p, dst_p, n_pad, ng1)
        h = _mlp(agg, h, Wa, ba, Wb, bb, block_rows)

    agg = _sc_gather_scatter_add(h, src_p, dst_p, n_pad, ng1)
    return _mlp_pool_classify(agg, h, W3a, b3a, W3b, b3b, batch3,
                              Wc1, bc1, Wc2, bc2, num_graphs, block_rows)


# speedup vs baseline: 1.3066x; 1.0027x over previous
import functools

import jax
import jax.numpy as jnp
from jax import lax
from jax.experimental import pallas as pl
from jax.experimental.pallas import tpu as pltpu
from jax.experimental.pallas import tpu_sc as plsc

_NC = 2
_NS = 16
_C = 128
_G = 8


def _sc_gather_scatter_add(x, src_p, dst_p, zero, n_pad, ng1):
    ns, NGT, G, C = src_p.shape
    N, D = x.shape
    ng0 = NGT - ng1
    assert ng0 % 2 == 0 and ng1 % 2 == 0 and G >= 4
    rows_init = (N // (_NS * 8)) * 8
    init_tail = N - _NS * rows_init
    rows_out = n_pad // _NS
    mesh = plsc.VectorSubcoreMesh(core_axis_name="c", subcore_axis_name="s")

    @functools.partial(
        pl.kernel,
        mesh=mesh,
        out_type=jax.ShapeDtypeStruct((_NC, n_pad, D), jnp.float32),
        scratch_types=[
            pltpu.VMEM((G, C), jnp.int32),
            pltpu.VMEM((G, C), jnp.int32),
            pltpu.VMEM((G, C), jnp.int32),
            pltpu.VMEM((G, C), jnp.int32),
            pltpu.VMEM((C, D), jnp.float32),
            pltpu.VMEM((C, D), jnp.float32),
            pltpu.VMEM_SHARED((n_pad, D), jnp.float32),
            pltpu.SemaphoreType.DMA,
            pltpu.SemaphoreType.DMA,
            pltpu.SemaphoreType.DMA,
            pltpu.SemaphoreType.DMA,
        ],
    )
    def k(x_hbm, src_hbm, dst_hbm, zero_hbm, out_hbm, sgA, sgB, dgA, dgB,
          rows0, rows1, acc, gsem0, gsem1, isemA, isemB):
        c = lax.axis_index("c")
        s = lax.axis_index("s")
        ng_c = jnp.where(c == 0, ng0, ng1)
        ngp = ng_c // 2
        gofs = jnp.where(c == 0, ng1, 0)
        pltpu.sync_copy(src_hbm.at[s, gofs], sgA)
        pltpu.sync_copy(dst_hbm.at[s, gofs], dgA)
        pltpu.async_copy(src_hbm.at[s, gofs + 1], sgB, isemB)
        pltpu.async_copy(dst_hbm.at[s, gofs + 1], dgB, isemB)
        pltpu.async_copy(x_hbm.at[sgA.at[0]], rows0, gsem0)
        pltpu.async_copy(x_hbm.at[sgA.at[1]], rows1, gsem1)
        @pl.when(c == 0)
        def _():
            pltpu.sync_copy(x_hbm.at[pl.ds(s * rows_init, rows_init)],
                            acc.at[pl.ds(s * rows_init, rows_init)])
            if init_tail:
                @pl.when(s == 0)
                def _():
                    pltpu.sync_copy(
                        x_hbm.at[pl.ds(_NS * rows_init, init_tail)],
                        acc.at[pl.ds(_NS * rows_init, init_tail)])

        @pl.when(c == 1)
        def _():
            pltpu.sync_copy(zero_hbm.at[pl.ds(s * rows_init, rows_init)],
                            acc.at[pl.ds(s * rows_init, rows_init)])
            if init_tail:
                @pl.when(s == 0)
                def _():
                    pltpu.sync_copy(
                        zero_hbm.at[pl.ds(_NS * rows_init, init_tail)],
                        acc.at[pl.ds(_NS * rows_init, init_tail)])

        plsc.subcore_barrier()

        @pl.loop(0, ngp)
        def pair(p):
            gA = gofs + 2 * p
            gB = gA + 1
            more = 2 * p + 2 < ng_c

            @pl.loop(0, G - 2, step=2)
            def _(j):
                pltpu.make_async_copy(x_hbm.at[sgA.at[j]], rows0, gsem0).wait()
                pltpu.sync_copy(rows0, acc.at[dgA.at[j]], add=True)
                pltpu.async_copy(x_hbm.at[sgA.at[j + 2]], rows0, gsem0)
                pltpu.make_async_copy(x_hbm.at[sgA.at[j + 1]], rows1,
                                      gsem1).wait()
                pltpu.sync_copy(rows1, acc.at[dgA.at[j + 1]], add=True)
                pltpu.async_copy(x_hbm.at[sgA.at[j + 3]], rows1, gsem1)

            pltpu.make_async_copy(src_hbm.at[s, gB], sgB, isemB).wait()
            pltpu.make_async_copy(dst_hbm.at[s, gB], dgB, isemB).wait()
            pltpu.make_async_copy(x_hbm.at[sgA.at[G - 2]], rows0, gsem0).wait()
            pltpu.sync_copy(rows0, acc.at[dgA.at[G - 2]], add=True)
            pltpu.async_copy(x_hbm.at[sgB.at[0]], rows0, gsem0)
            pltpu.make_async_copy(x_hbm.at[sgA.at[G - 1]], rows1, gsem1).wait()
            pltpu.sync_copy(rows1, acc.at[dgA.at[G - 1]], add=True)
            pltpu.async_copy(x_hbm.at[sgB.at[1]], rows1, gsem1)

            @pl.when(more)
            def _():
                pltpu.async_copy(src_hbm.at[s, gA + 2], sgA, isemA)
                pltpu.async_copy(dst_hbm.at[s, gA + 2], dgA, isemA)

            @pl.loop(0, G - 2, step=2)
            def _(j):
                pltpu.make_async_copy(x_hbm.at[sgB.at[j]], rows0, gsem0).wait()
                pltpu.sync_copy(rows0, acc.at[dgB.at[j]], add=True)
                pltpu.async_copy(x_hbm.at[sgB.at[j + 2]], rows0, gsem0)
                pltpu.make_async_copy(x_hbm.at[sgB.at[j + 1]], rows1,
                                      gsem1).wait()
                pltpu.sync_copy(rows1, acc.at[dgB.at[j + 1]], add=True)
                pltpu.async_copy(x_hbm.at[sgB.at[j + 3]], rows1, gsem1)

            @pl.when(more)
            def _():
                pltpu.make_async_copy(src_hbm.at[s, gA + 2], sgA, isemA).wait()
                pltpu.make_async_copy(dst_hbm.at[s, gA + 2], dgA, isemA).wait()

            pltpu.make_async_copy(x_hbm.at[sgB.at[G - 2]], rows0, gsem0).wait()
            pltpu.sync_copy(rows0, acc.at[dgB.at[G - 2]], add=True)

            @pl.when(more)
            def _():
                pltpu.async_copy(x_hbm.at[sgA.at[0]], rows0, gsem0)

            pltpu.make_async_copy(x_hbm.at[sgB.at[G - 1]], rows1, gsem1).wait()
            pltpu.sync_copy(rows1, acc.at[dgB.at[G - 1]], add=True)

            @pl.when(more)
            def _():
                pltpu.async_copy(x_hbm.at[sgA.at[1]], rows1, gsem1)

            @pl.when(2 * p + 3 < ng_c)
            def _():
                pltpu.async_copy(src_hbm.at[s, gB + 2], sgB, isemB)
                pltpu.async_copy(dst_hbm.at[s, gB + 2], dgB, isemB)

        plsc.subcore_barrier()
        pltpu.sync_copy(acc.at[pl.ds(s * rows_out, rows_out)],
                        out_hbm.at[c, pl.ds(s * rows_out, rows_out)])

    return k(x, src_p, dst_p, zero)


def _mlp(agg, n_rows, Wa, ba, Wb, bb, block_rows):
    D = agg.shape[2]
    grid = n_rows // block_rows

    def body(agg_ref, wa, ba_r, wb, bb_r, o_ref):
        g = agg_ref[0] + agg_ref[1]
        h1 = jnp.maximum(
            jnp.dot(g, wa[...], preferred_element_type=jnp.float32)
            + ba_r[...], 0.0)
        o_ref[...] = (jnp.dot(h1, wb[...], preferred_element_type=jnp.float32)
                      + bb_r[...])

    return pl.pallas_call(
        body,
        grid=(grid,),
        in_specs=[
            pl.BlockSpec((_NC, block_rows, D), lambda i: (0, i, 0)),
            pl.BlockSpec((D, D), lambda i: (0, 0)),
            pl.BlockSpec((1, D), lambda i: (0, 0)),
            pl.BlockSpec((D, D), lambda i: (0, 0)),
            pl.BlockSpec((1, D), lambda i: (0, 0)),
        ],
        out_specs=pl.BlockSpec((block_rows, D), lambda i: (i, 0)),
        out_shape=jax.ShapeDtypeStruct((n_rows, D), jnp.float32),
    )(agg, Wa, ba.reshape(1, D), Wb, bb.reshape(1, D))


def _mlp_pool_classify(agg, n_rows, Wa, ba, Wb, bb, batch3, Wc1, bc1, Wc2,
                       bc2, num_graphs, block_rows):
    N = n_rows
    D = agg.shape[2]
    n_classes = Wc2.shape[1]
    grid = N // block_rows

    def body(agg_ref, b_ref, wa, ba_r, wb, bb_r, wc1, bc1_r, wc2,
             bc2_r, o_ref, acc_ref):
        i = pl.program_id(0)

        @pl.when(i == 0)
        def _():
            acc_ref[...] = jnp.zeros_like(acc_ref)

        g = agg_ref[0] + agg_ref[1]
        h1 = jnp.maximum(
            jnp.dot(g, wa[...], preferred_element_type=jnp.float32)
            + ba_r[...], 0.0)
        h3 = (jnp.dot(h1, wb[...], preferred_element_type=jnp.float32)
              + bb_r[...])
        b = b_ref[0, 0, :]
        onehot = (b[:, None] == lax.broadcasted_iota(
            jnp.int32, (block_rows, num_graphs), 1)).astype(jnp.float32)
        acc_ref[...] += lax.dot_general(
            onehot, h3, (((0,), (0,)), ((), ())),
            preferred_element_type=jnp.float32)

        @pl.when(i == grid - 1)
        def _():
            t = jnp.maximum(
                jnp.dot(acc_ref[...], wc1[...],
                        preferred_element_type=jnp.float32) + bc1_r[...], 0.0)
            o_ref[...] = (jnp.dot(t, wc2[...],
                                  preferred_element_type=jnp.float32)
                          + bc2_r[...])

    return pl.pallas_call(
        body,
        grid=(grid,),
        in_specs=[
            pl.BlockSpec((_NC, block_rows, D), lambda i: (0, i, 0)),
            pl.BlockSpec((1, 1, block_rows), lambda i: (i, 0, 0)),
            pl.BlockSpec((D, D), lambda i: (0, 0)),
            pl.BlockSpec((1, D), lambda i: (0, 0)),
            pl.BlockSpec((D, D), lambda i: (0, 0)),
            pl.BlockSpec((1, D), lambda i: (0, 0)),
            pl.BlockSpec((D, D), lambda i: (0, 0)),
            pl.BlockSpec((1, D), lambda i: (0, 0)),
            pl.BlockSpec((D, n_classes), lambda i: (0, 0)),
            pl.BlockSpec((1, n_classes), lambda i: (0, 0)),
        ],
        out_specs=pl.BlockSpec((num_graphs, n_classes), lambda i: (0, 0)),
        out_shape=jax.ShapeDtypeStruct((num_graphs, n_classes), jnp.float32),
        scratch_shapes=[pltpu.VMEM((num_graphs, D), jnp.float32)],
    )(agg, batch3, Wa, ba.reshape(1, D), Wb, bb.reshape(1, D),
      Wc1, bc1.reshape(1, D), Wc2, bc2.reshape(1, n_classes))


def kernel(x, edge_index, batch, W1a, b1a, W1b, b1b, W2a, b2a, W2b, b2b,
           W3a, b3a, W3b, b3b, Wc1, bc1, Wc2, bc2):
    N, D = x.shape
    E = edge_index.shape[1]
    num_graphs = 64
    per_slab = -(-E // _NS)
    chunks_per_slab = -(-per_slab // _C)
    NGT = -(-chunks_per_slab // _G)
    ng1 = NGT // 2
    ng1 = ng1 - (ng1 % 2)
    slab_cap = NGT * _G * _C
    n_pad = -(-(N + 1) // (_NS * 8)) * _NS * 8

    src = edge_index[0]
    dst = edge_index[1]
    if E % _NS == 0:
        pad = slab_cap - E // _NS
        fill = (jnp.arange(_NS * pad, dtype=jnp.int32) % N).reshape(_NS, pad)
        src_p = jnp.concatenate([src.reshape(_NS, E // _NS), fill], axis=1)
        dst_p = jnp.pad(dst.reshape(_NS, E // _NS), ((0, 0), (0, pad)),
                        constant_values=N)
    else:
        pad = _NS * slab_cap - E
        fill = jnp.arange(pad, dtype=jnp.int32) % N
        src_p = jnp.concatenate([src, fill])
        dst_p = jnp.concatenate([dst, jnp.full((pad,), N, jnp.int32)])
    src_p = src_p.reshape(_NS, NGT, _G, _C)
    dst_p = dst_p.reshape(_NS, NGT, _G, _C)

    block_rows = 2000
    batch3 = batch.reshape(N // block_rows, 1, block_rows)

    zero = jnp.zeros((N, D), jnp.float32)
    h = x
    for (Wa, ba, Wb, bb) in ((W1a, b1a, W1b, b1b), (W2a, b2a, W2b, b2b)):
        agg = _sc_gather_scatter_add(h, src_p, dst_p, zero, n_pad, ng1)
        h = _mlp(agg, N, Wa, ba, Wb, bb, block_rows)

    agg = _sc_gather_scatter_add(h, src_p, dst_p, zero, n_pad, ng1)
    return _mlp_pool_classify(agg, N, W3a, b3a, W3b, b3b, batch3,
                              Wc1, bc1, Wc2, bc2, num_graphs, block_rows)
